# baseline (speedup 1.0000x reference)
import functools

import jax
import jax.numpy as jnp
from jax import lax
from jax.experimental import pallas as pl
from jax.experimental.pallas import tpu as pltpu
from jax.experimental.pallas import tpu_sc as plsc

VOCAB = 100000
EMBED_DIM = 16
BATCH = 1024

VBLK = 1024
V_PAD = 100352
NV = V_PAD // VBLK
NFULL = NV - 1
TAIL = VOCAB - NFULL * VBLK
NBUF = 4
KDIM = 24
LANES = 128
NLG = VBLK // LANES
NR = 4
RCH = BATCH // NR

LOG2E = 1.4426950408889634
LN2 = 0.6931471805599453

SC_CORES = 2
SC_SUBCORES = 16
NW = SC_CORES * SC_SUBCORES
B_PER_W = BATCH // NW


def _make_sc_gather():
    mesh = plsc.VectorSubcoreMesh(core_axis_name="c", subcore_axis_name="s")

    @functools.partial(
        pl.kernel,
        mesh=mesh,
        out_type=jax.ShapeDtypeStruct((BATCH, EMBED_DIM), jnp.float32),
        scratch_types=[
            pltpu.VMEM((B_PER_W,), jnp.int32),
            pltpu.VMEM((B_PER_W, EMBED_DIM), jnp.float32),
            pltpu.SemaphoreType.DMA,
        ],
        compiler_params=pltpu.CompilerParams(use_tc_tiling_on_sc=False),
    )
    def gather_kernel(table_hbm, idx_hbm, out_hbm, idx_v, rows_v, sem):
        wid = lax.axis_index("s") * SC_CORES + lax.axis_index("c")
        base = wid * B_PER_W
        pltpu.sync_copy(idx_hbm.at[pl.ds(base, B_PER_W)], idx_v)
        pltpu.async_copy(table_hbm.at[idx_v], rows_v, sem).wait()
        pltpu.sync_copy(rows_v, out_hbm.at[pl.ds(base, B_PER_W)])

    return gather_kernel


def _fused_kernel(eb_ref, wts_ref, wto_ref, o_hbm,
                  m_ref, s_ref, escr_ref, buf_ref, tail_ref, sem, tail_sem):
    r = pl.program_id(0)
    p = pl.program_id(1)
    j = pl.program_id(2)

    @pl.when(p == 0)
    def _stats():
        logits2 = jnp.dot(eb_ref[...], wts_ref[...],
                          preferred_element_type=jnp.float32)

        @pl.when(j == 0)
        def _():
            m_ref[...] = jnp.full((RCH, LANES), -3.0e38, jnp.float32)
            s_ref[...] = jnp.zeros((RCH, LANES), jnp.float32)

        parts = [logits2[:, k * LANES:(k + 1) * LANES] for k in range(NLG)]
        m_old = m_ref[...]
        m_new = m_old
        for q in parts:
            m_new = jnp.maximum(m_new, q)
        s = s_ref[...] * jnp.exp2(m_old - m_new)
        for q in parts:
            s = s + jnp.exp2(q - m_new)
        m_ref[...] = m_new
        s_ref[...] = s

        @pl.when(j == NV - 1)
        def _():
            m_fin = jnp.max(m_new, axis=1, keepdims=True)
            s_fin = jnp.sum(s * jnp.exp2(m_new - m_fin), axis=1, keepdims=True)
            lse = (m_fin + jnp.log2(s_fin)) * LN2
            e = eb_ref[...]
            escr_ref[...] = jnp.concatenate(
                [e[:, :EMBED_DIM + 1], -lse,
                 jnp.zeros((RCH, KDIM - EMBED_DIM - 2), jnp.float32)], axis=1)

    @pl.when(p == 1)
    def _out():
        logits_t = lax.dot_general(wto_ref[...], escr_ref[...],
                                   (((0,), (1,)), ((), ())),
                                   preferred_element_type=jnp.float32)
        os_ = r * NFULL + j
        slot = lax.rem(os_, NBUF)

        @pl.when(j < NFULL)
        def _():
            @pl.when(os_ >= NBUF)
            def _():
                os_prev = os_ - NBUF
                jp = lax.rem(os_prev, NFULL)
                rp = os_prev // NFULL
                pltpu.make_async_copy(
                    buf_ref.at[slot],
                    o_hbm.at[pl.ds(jp * VBLK, VBLK), pl.ds(rp * RCH, RCH)],
                    sem.at[slot],
                ).wait()

            buf_ref[slot] = logits_t
            pltpu.make_async_copy(
                buf_ref.at[slot],
                o_hbm.at[pl.ds(j * VBLK, VBLK), pl.ds(r * RCH, RCH)],
                sem.at[slot],
            ).start()

        @pl.when(j == NV - 1)
        def _():
            @pl.when(r > 0)
            def _():
                pltpu.make_async_copy(
                    tail_ref,
                    o_hbm.at[pl.ds(NFULL * VBLK, TAIL),
                             pl.ds((r - 1) * RCH, RCH)],
                    tail_sem,
                ).wait()

            tail_ref[...] = logits_t[:TAIL, :]
            pltpu.make_async_copy(
                tail_ref,
                o_hbm.at[pl.ds(NFULL * VBLK, TAIL), pl.ds(r * RCH, RCH)],
                tail_sem,
            ).start()

            @pl.when(r == NR - 1)
            def _():
                for d in range(NBUF):
                    osd = NR * NFULL - 1 - d
                    jd, rd = osd % NFULL, osd // NFULL
                    pltpu.make_async_copy(
                        buf_ref.at[osd % NBUF],
                        o_hbm.at[pl.ds(jd * VBLK, VBLK),
                                 pl.ds(rd * RCH, RCH)],
                        sem.at[osd % NBUF],
                    ).wait()
                pltpu.make_async_copy(
                    tail_ref,
                    o_hbm.at[pl.ds(NFULL * VBLK, TAIL),
                             pl.ds((NR - 1) * RCH, RCH)],
                    tail_sem,
                ).wait()


def kernel(inputs, emb_table, W, b):
    idx = inputs.astype(jnp.int32)
    embeds = _make_sc_gather()(emb_table, idx)

    ones_col = jnp.ones((BATCH, 1), jnp.float32)
    e_base = jnp.concatenate(
        [embeds, ones_col,
         jnp.zeros((BATCH, KDIM - EMBED_DIM - 1), jnp.float32)], axis=1)

    wt = W.T
    pad = V_PAD - VOCAB
    wt_s = jnp.concatenate([
        jnp.pad(wt * LOG2E, ((0, 0), (0, pad))),
        jnp.pad(b * LOG2E, (0, pad), constant_values=-1e30).reshape(1, V_PAD),
        jnp.zeros((KDIM - EMBED_DIM - 1, V_PAD), jnp.float32),
    ], axis=0)
    wt_o = jnp.concatenate([
        jnp.pad(wt, ((0, 0), (0, pad))),
        jnp.pad(b, (0, pad)).reshape(1, V_PAD),
        jnp.ones((1, V_PAD), jnp.float32),
        jnp.zeros((KDIM - EMBED_DIM - 2, V_PAD), jnp.float32),
    ], axis=0)

    out_t = pl.pallas_call(
        _fused_kernel,
        grid=(NR, 2, NV),
        in_specs=[
            pl.BlockSpec((RCH, KDIM), lambda r, p, j: (r, 0)),
            pl.BlockSpec((KDIM, VBLK),
                         lambda r, p, j: (0, jnp.where(p == 0, j, 0))),
            pl.BlockSpec((KDIM, VBLK),
                         lambda r, p, j: (0, jnp.where(p == 1, j, 0))),
        ],
        out_specs=pl.BlockSpec(memory_space=pl.ANY),
        out_shape=jax.ShapeDtypeStruct((VOCAB, BATCH), jnp.float32),
        scratch_shapes=[
            pltpu.VMEM((RCH, LANES), jnp.float32),
            pltpu.VMEM((RCH, LANES), jnp.float32),
            pltpu.VMEM((RCH, KDIM), jnp.float32),
            pltpu.VMEM((NBUF, VBLK, RCH), jnp.float32),
            pltpu.VMEM((TAIL, RCH), jnp.float32),
            pltpu.SemaphoreType.DMA((NBUF,)),
            pltpu.SemaphoreType.DMA,
        ],
    )(e_base, wt_s, wt_o)
    return out_t.T


# speedup vs baseline: 1.9244x; 1.9244x over previous
import functools

import jax
import jax.numpy as jnp
from jax import lax
from jax.experimental import pallas as pl
from jax.experimental.pallas import tpu as pltpu
from jax.experimental.pallas import tpu_sc as plsc

VOCAB = 100000
EMBED_DIM = 16
BATCH = 1024

V_PAD = 100352
SBLK = 2048
NSV = V_PAD // SBLK
OBLK = 2048
NOV = V_PAD // OBLK
TAIL = VOCAB - (NOV - 1) * OBLK
NBUF = 4
KDIM = 24
LANES = 128
NLG = SBLK // LANES

LOG2E = 1.4426950408889634
LN2 = 0.6931471805599453

SC_CORES = 2
SC_SUBCORES = 16
NW = SC_CORES * SC_SUBCORES
B_PER_W = BATCH // NW


def _make_sc_gather():
    mesh = plsc.VectorSubcoreMesh(core_axis_name="c", subcore_axis_name="s")

    @functools.partial(
        pl.kernel,
        mesh=mesh,
        out_type=jax.ShapeDtypeStruct((BATCH, EMBED_DIM), jnp.float32),
        scratch_types=[
            pltpu.VMEM((B_PER_W,), jnp.int32),
            pltpu.VMEM((B_PER_W, EMBED_DIM), jnp.float32),
            pltpu.SemaphoreType.DMA,
        ],
        compiler_params=pltpu.CompilerParams(use_tc_tiling_on_sc=False),
    )
    def gather_kernel(table_hbm, idx_hbm, out_hbm, idx_v, rows_v, sem):
        wid = lax.axis_index("s") * SC_CORES + lax.axis_index("c")
        base = wid * B_PER_W
        pltpu.sync_copy(idx_hbm.at[pl.ds(base, B_PER_W)], idx_v)
        pltpu.async_copy(table_hbm.at[idx_v], rows_v, sem).wait()
        pltpu.sync_copy(rows_v, out_hbm.at[pl.ds(base, B_PER_W)])

    return gather_kernel


def _stats_kernel(e_ref, wt_ref, lse_ref, m_ref, s_ref):
    j = pl.program_id(0)
    logits2 = jnp.dot(e_ref[...], wt_ref[...],
                      preferred_element_type=jnp.float32)

    @pl.when(j == 0)
    def _():
        m_ref[...] = jnp.full((BATCH, LANES), -3.0e38, jnp.float32)
        s_ref[...] = jnp.zeros((BATCH, LANES), jnp.float32)

    parts = [logits2[:, k * LANES:(k + 1) * LANES] for k in range(NLG)]
    m_old = m_ref[...]
    m_new = m_old
    for p in parts:
        m_new = jnp.maximum(m_new, p)
    s = s_ref[...] * jnp.exp2(m_old - m_new)
    for p in parts:
        s = s + jnp.exp2(p - m_new)
    m_ref[...] = m_new
    s_ref[...] = s

    @pl.when(j == pl.num_programs(0) - 1)
    def _():
        m_fin = jnp.max(m_new, axis=1, keepdims=True)
        s_fin = jnp.sum(s * jnp.exp2(m_new - m_fin), axis=1, keepdims=True)
        lse_ref[...] = (m_fin + jnp.log2(s_fin)) * LN2


def _out_kernel(e_ref, wt_ref, o_hbm, buf_ref, tail_ref, sem, tail_sem):
    j = pl.program_id(0)
    slot = lax.rem(j, NBUF)

    @pl.when(j >= NBUF)
    def _():
        pltpu.make_async_copy(
            buf_ref.at[slot],
            o_hbm.at[pl.ds((j - NBUF) * OBLK, OBLK), :],
            sem.at[slot],
        ).wait()

    logits_t = lax.dot_general(wt_ref[...], e_ref[...],
                               (((0,), (1,)), ((), ())),
                               preferred_element_type=jnp.float32)

    @pl.when(j < NOV - 1)
    def _():
        buf_ref[slot] = logits_t
        pltpu.make_async_copy(
            buf_ref.at[slot],
            o_hbm.at[pl.ds(j * OBLK, OBLK), :],
            sem.at[slot],
        ).start()

    @pl.when(j == NOV - 1)
    def _():
        tail_ref[...] = logits_t[:TAIL, :]
        pltpu.make_async_copy(
            tail_ref,
            o_hbm.at[pl.ds((NOV - 1) * OBLK, TAIL), :],
            tail_sem,
        ).start()
        for d in range(NBUF - 1, 0, -1):
            jj = NOV - 1 - d
            pltpu.make_async_copy(
                buf_ref.at[jj % NBUF],
                o_hbm.at[pl.ds(jj * OBLK, OBLK), :],
                sem.at[jj % NBUF],
            ).wait()
        pltpu.make_async_copy(
            tail_ref,
            o_hbm.at[pl.ds((NOV - 1) * OBLK, TAIL), :],
            tail_sem,
        ).wait()


def kernel(inputs, emb_table, W, b):
    idx = inputs.astype(jnp.int32)
    embeds = _make_sc_gather()(emb_table, idx)

    ones_col = jnp.ones((BATCH, 1), jnp.float32)
    zeros_cols = jnp.zeros((BATCH, KDIM - EMBED_DIM - 1), jnp.float32)
    e_stats = jnp.concatenate([embeds, ones_col, zeros_cols], axis=1)

    wt = W.T
    pad = V_PAD - VOCAB
    wt_s = jnp.concatenate([
        jnp.pad(wt * LOG2E, ((0, 0), (0, pad))),
        jnp.pad(b * LOG2E, (0, pad), constant_values=-1e30).reshape(1, V_PAD),
        jnp.zeros((KDIM - EMBED_DIM - 1, V_PAD), jnp.float32),
    ], axis=0)

    lse = pl.pallas_call(
        _stats_kernel,
        grid=(NSV,),
        in_specs=[
            pl.BlockSpec((BATCH, KDIM), lambda j: (0, 0)),
            pl.BlockSpec((KDIM, SBLK), lambda j: (0, j)),
        ],
        out_specs=pl.BlockSpec((BATCH, 1), lambda j: (0, 0)),
        out_shape=jax.ShapeDtypeStruct((BATCH, 1), jnp.float32),
        scratch_shapes=[
            pltpu.VMEM((BATCH, LANES), jnp.float32),
            pltpu.VMEM((BATCH, LANES), jnp.float32),
        ],
    )(e_stats, wt_s)

    e_out = jnp.concatenate(
        [embeds, ones_col, -lse,
         jnp.zeros((BATCH, KDIM - EMBED_DIM - 2), jnp.float32)], axis=1)
    wt_o = jnp.concatenate([
        jnp.pad(wt, ((0, 0), (0, pad))),
        jnp.pad(b, (0, pad)).reshape(1, V_PAD),
        jnp.ones((1, V_PAD), jnp.float32),
        jnp.zeros((KDIM - EMBED_DIM - 2, V_PAD), jnp.float32),
    ], axis=0)

    out_t = pl.pallas_call(
        _out_kernel,
        grid=(NOV,),
        in_specs=[
            pl.BlockSpec((BATCH, KDIM), lambda j: (0, 0)),
            pl.BlockSpec((KDIM, OBLK), lambda j: (0, j)),
        ],
        out_specs=pl.BlockSpec(memory_space=pl.ANY),
        out_shape=jax.ShapeDtypeStruct((VOCAB, BATCH), jnp.float32),
        scratch_shapes=[
            pltpu.VMEM((NBUF, OBLK, BATCH), jnp.float32),
            pltpu.VMEM((TAIL, BATCH), jnp.float32),
            pltpu.SemaphoreType.DMA((NBUF,)),
            pltpu.SemaphoreType.DMA,
        ],
    )(e_out, wt_o)
    return out_t.T
